# baseline (speedup 1.0000x reference)
import functools
import jax
import jax.numpy as jnp
from jax import lax
from jax.experimental import pallas as pl
from jax.experimental.pallas import tpu as pltpu

WSZ = 8
NUM_HEADS = 2
CR = 2
FILTER_RATE = 0.3
NUM_BLOCKS = 2
C = 96
T = 16


def _gelu(x):
    return 0.5 * x * (1.0 + lax.erf(x * (2.0 ** -0.5)))


def _compute_body(b_ref, h_ref, w_ref, *refs):
    (xw_refs, (ng, nb, lw, lb, qw, qb, pw, pb, o_ref)) = refs[:T], refs[T:]
    winsz = WSZ * WSZ
    xs = [r[...].reshape(winsz, C) for r in xw_refs]
    x0 = jnp.concatenate(xs, axis=0)
    mu = jnp.mean(x0, axis=-1, keepdims=True)
    var = jnp.mean((x0 - mu) * (x0 - mu), axis=-1, keepdims=True)
    xln = (x0 - mu) * lax.rsqrt(var + 1e-5) * ng[...] + nb[...]
    h = _gelu(
        jnp.dot(xln, lw[...], preferred_element_type=jnp.float32) + lb[...])
    qkv = jnp.dot(h, qw[...], preferred_element_type=jnp.float32) + qb[...]
    C2 = C * CR
    hd = C2 // NUM_HEADS
    scale = (C // NUM_HEADS) ** (-0.5)
    outs = []
    for j in range(T):
        r0 = j * winsz
        head_outs = []
        for hh in range(NUM_HEADS):
            c0 = hh * hd
            qh = qkv[r0:r0 + winsz, c0:c0 + hd]
            kh = qkv[r0:r0 + winsz, C2 + c0:C2 + c0 + hd]
            vh = qkv[r0:r0 + winsz, 2 * C2 + c0:2 * C2 + c0 + hd]
            s = lax.dot_general(qh, kh, (((1,), (1,)), ((), ())),
                                preferred_element_type=jnp.float32) * scale
            s = s - jnp.max(s, axis=-1, keepdims=True)
            e = jnp.exp(s)
            p = e / jnp.sum(e, axis=-1, keepdims=True)
            head_outs.append(
                jnp.dot(p, vh, preferred_element_type=jnp.float32))
        outs.append(jnp.concatenate(head_outs, axis=1))
    attn = jnp.concatenate(outs, axis=0)
    xf2 = h + attn
    proj = _gelu(
        jnp.dot(xf2, pw[...], preferred_element_type=jnp.float32) + pb[...])
    o_ref[...] = x0 + xln + proj


def _scatter_body(b_ref, h_ref, w_ref, rows_ref, xw_ref, o_ref):
    o_ref[...] = rows_ref[...].reshape(1, 1, WSZ, 1, WSZ, C)


def _refine_block_pallas(x_nhwc, unc, shift_direct, p):
    wsz = WSZ
    s2 = wsz // 2
    if shift_direct == 1:
        x_nhwc = jnp.pad(x_nhwc, ((0, 0), (0, 0), (s2, s2), (0, 0)))
        unc = jnp.pad(unc, ((0, 0), (0, 0), (s2, s2)))
    elif shift_direct == 2:
        x_nhwc = jnp.pad(x_nhwc, ((0, 0), (s2, s2), (0, 0), (0, 0)))
        unc = jnp.pad(unc, ((0, 0), (s2, s2), (0, 0)))
    elif shift_direct == 3:
        x_nhwc = jnp.pad(x_nhwc, ((0, 0), (s2, s2), (s2, s2), (0, 0)))
        unc = jnp.pad(unc, ((0, 0), (s2, s2), (s2, s2)))
    B, H, W, Cc = x_nhwc.shape
    nH, nW = H // wsz, W // wsz
    nWin = nH * nW
    winsz = wsz * wsz
    nWF = int(nWin * FILTER_RATE)

    score = unc.reshape(B, nH, wsz, nW, wsz).mean(axis=(2, 4))
    _, idx = lax.top_k(score.reshape(B, nWin), nWF)
    idx = (idx + jnp.arange(B, dtype=idx.dtype)[:, None] * nWin).reshape(-1)
    S = B * nWF
    Sp = ((S + T - 1) // T) * T
    idx_p = jnp.concatenate(
        [idx, jnp.broadcast_to(idx[-1:], (Sp - S,))]) if Sp > S else idx
    b_arr = (idx_p // nWin).astype(jnp.int32)
    r_arr = idx_p % nWin
    h_arr = (r_arr // nW).astype(jnp.int32)
    w_arr = (r_arr % nW).astype(jnp.int32)

    xw6 = x_nhwc.reshape(B, nH, wsz, nW, wsz, Cc)

    def win_map(i, b, h, w, j=0):
        return (b[i * T + j], h[i * T + j], 0, w[i * T + j], 0, 0)

    win_specs = [
        pl.BlockSpec((1, 1, wsz, 1, wsz, Cc), functools.partial(win_map, j=j))
        for j in range(T)
    ]
    C2 = Cc * CR

    def const2(i, b, h, w):
        return (0, 0)

    wt_specs = [
        pl.BlockSpec((1, Cc), const2),
        pl.BlockSpec((1, Cc), const2),
        pl.BlockSpec((Cc, C2), const2),
        pl.BlockSpec((1, C2), const2),
        pl.BlockSpec((C2, 3 * C2), const2),
        pl.BlockSpec((1, 3 * C2), const2),
        pl.BlockSpec((C2, Cc), const2),
        pl.BlockSpec((1, Cc), const2),
    ]
    compact = pl.pallas_call(
        _compute_body,
        grid_spec=pltpu.PrefetchScalarGridSpec(
            num_scalar_prefetch=3,
            grid=(Sp // T,),
            in_specs=win_specs + wt_specs,
            out_specs=pl.BlockSpec((T * winsz, Cc), lambda i, b, h, w: (i, 0)),
        ),
        out_shape=jax.ShapeDtypeStruct((Sp * winsz, Cc), jnp.float32),
    )(b_arr, h_arr, w_arr, *([xw6] * T),
      p['norm_g'].reshape(1, Cc), p['norm_b'].reshape(1, Cc),
      p['lin_w'], p['lin_b'].reshape(1, C2),
      p['qkv_w'], p['qkv_b'].reshape(1, 3 * C2),
      p['proj_w'], p['proj_b'].reshape(1, Cc))

    xw6_new = pl.pallas_call(
        _scatter_body,
        grid_spec=pltpu.PrefetchScalarGridSpec(
            num_scalar_prefetch=3,
            grid=(S,),
            in_specs=[
                pl.BlockSpec((winsz, Cc), lambda i, b, h, w: (i, 0)),
                pl.BlockSpec(memory_space=pl.ANY),
            ],
            out_specs=pl.BlockSpec(
                (1, 1, wsz, 1, wsz, Cc),
                lambda i, b, h, w: (b[i], h[i], 0, w[i], 0, 0)),
        ),
        out_shape=jax.ShapeDtypeStruct(xw6.shape, jnp.float32),
        input_output_aliases={4: 0},
    )(b_arr, h_arr, w_arr, compact, xw6)

    x_nhwc = xw6_new.reshape(B, H, W, Cc)
    if shift_direct == 1:
        x_nhwc = x_nhwc[:, :, s2:-s2, :]
    elif shift_direct == 2:
        x_nhwc = x_nhwc[:, s2:-s2, :, :]
    elif shift_direct == 3:
        x_nhwc = x_nhwc[:, s2:-s2, s2:-s2, :]
    return x_nhwc


def _sep_conv_bn_relu6_nhwc(x, dw, pw, g, b, m, v):
    dwt = jnp.transpose(dw, (2, 3, 1, 0))
    pwt = jnp.transpose(pw, (2, 3, 1, 0))
    y = lax.conv_general_dilated(
        x, dwt, (1, 1), [(3, 3), (3, 3)],
        dimension_numbers=('NHWC', 'HWIO', 'NHWC'),
        feature_group_count=x.shape[-1])
    y = lax.conv_general_dilated(
        y, pwt, (1, 1), [(0, 0), (0, 0)],
        dimension_numbers=('NHWC', 'HWIO', 'NHWC'))
    y = (y - m) * lax.rsqrt(v + 1e-5)
    y = y * g + b
    return jnp.clip(y, 0.0, 6.0)


def kernel(feature_map, coarse_pred, params):
    x = jnp.transpose(feature_map, (0, 2, 3, 1))
    x = _sep_conv_bn_relu6_nhwc(
        x, params['conv_in_dw'], params['conv_in_pw'],
        params['conv_in_bn_g'], params['conv_in_bn_b'],
        params['conv_in_bn_m'], params['conv_in_bn_v'])
    probs = jax.nn.softmax(coarse_pred, axis=1)
    unc = 1.0 - probs.max(axis=1)
    for i in range(NUM_BLOCKS):
        if i != 0:
            x = _sep_conv_bn_relu6_nhwc(
                x, params['conv_%d_dw' % i], params['conv_%d_pw' % i],
                params['conv_%d_bn_g' % i], params['conv_%d_bn_b' % i],
                params['conv_%d_bn_m' % i], params['conv_%d_bn_v' % i])
        pre = 'blk%d_' % i
        p = {k[len(pre):]: v for k, v in params.items() if k.startswith(pre)}
        x = _refine_block_pallas(x, unc, i % 4, p)
    return jnp.transpose(x, (0, 3, 1, 2))


# speedup vs baseline: 1.3216x; 1.3216x over previous
import functools
import jax
import jax.numpy as jnp
from jax import lax
from jax.experimental import pallas as pl
from jax.experimental.pallas import tpu as pltpu

WSZ = 8
NUM_HEADS = 2
CR = 2
FILTER_RATE = 0.3
NUM_BLOCKS = 2
C = 96
T = 16
G = 4
PH = 128
PC2 = NUM_HEADS * PH


def _gelu(x):
    return 0.5 * x * (1.0 + lax.erf(x * (2.0 ** -0.5)))


def _bf(x):
    return x.astype(jnp.bfloat16)


def _compute_body(idx_ref, *refs):
    (xw_refs, (ng, nb, lw, lb, qw, qb, pw, pb, o_ref)) = refs[:T], refs[T:]
    winsz = WSZ * WSZ
    xs = [r[...].reshape(winsz, C) for r in xw_refs]
    x0 = jnp.concatenate(xs, axis=0)
    mu = jnp.mean(x0, axis=-1, keepdims=True)
    var = jnp.mean((x0 - mu) * (x0 - mu), axis=-1, keepdims=True)
    xln = (x0 - mu) * lax.rsqrt(var + 1e-5) * ng[...] + nb[...]
    h = _gelu(
        jnp.dot(_bf(xln), lw[...], preferred_element_type=jnp.float32)
        + lb[...])
    qkv = jnp.dot(_bf(h), qw[...], preferred_element_type=jnp.float32) \
        + qb[...]
    scale = (C // NUM_HEADS) ** (-0.5)
    qkvb = _bf(qkv)
    gw = G * winsz
    ri = lax.broadcasted_iota(jnp.int32, (gw, gw), 0) // winsz
    ci = lax.broadcasted_iota(jnp.int32, (gw, gw), 1) // winsz
    mask = ri == ci
    outs = []
    for g in range(T // G):
        r0 = g * gw
        head_outs = []
        for hh in range(NUM_HEADS):
            c0 = hh * PH
            qh = qkvb[r0:r0 + gw, c0:c0 + PH]
            kh = qkvb[r0:r0 + gw, PC2 + c0:PC2 + c0 + PH]
            vh = qkvb[r0:r0 + gw, 2 * PC2 + c0:2 * PC2 + c0 + PH]
            s = lax.dot_general(qh, kh, (((1,), (1,)), ((), ())),
                                preferred_element_type=jnp.float32) * scale
            s = jnp.where(mask, s, -1e30)
            s = s - jnp.max(s, axis=-1, keepdims=True)
            e = jnp.exp(s)
            p = e / jnp.sum(e, axis=-1, keepdims=True)
            head_outs.append(
                jnp.dot(_bf(p), vh, preferred_element_type=jnp.float32))
        outs.append(jnp.concatenate(head_outs, axis=1))
    attn = jnp.concatenate(outs, axis=0)
    xf2 = _bf(h + attn)
    proj = _gelu(
        jnp.dot(xf2, pw[...], preferred_element_type=jnp.float32) + pb[...])
    o_ref[...] = x0 + xln + proj


def _scatter_body(idx_ref, rows_ref, xw_ref, o_ref):
    o_ref[...] = rows_ref[...].reshape(1, WSZ * WSZ, C)


def _refine_block_pallas(x_nhwc, unc, shift_direct, p):
    wsz = WSZ
    s2 = wsz // 2
    if shift_direct == 1:
        x_nhwc = jnp.pad(x_nhwc, ((0, 0), (0, 0), (s2, s2), (0, 0)))
        unc = jnp.pad(unc, ((0, 0), (0, 0), (s2, s2)))
    elif shift_direct == 2:
        x_nhwc = jnp.pad(x_nhwc, ((0, 0), (s2, s2), (0, 0), (0, 0)))
        unc = jnp.pad(unc, ((0, 0), (s2, s2), (0, 0)))
    elif shift_direct == 3:
        x_nhwc = jnp.pad(x_nhwc, ((0, 0), (s2, s2), (s2, s2), (0, 0)))
        unc = jnp.pad(unc, ((0, 0), (s2, s2), (s2, s2)))
    B, H, W, Cc = x_nhwc.shape
    nH, nW = H // wsz, W // wsz
    nWin = nH * nW
    winsz = wsz * wsz
    nWF = int(nWin * FILTER_RATE)

    score = unc.reshape(B, nH, wsz, nW, wsz).mean(axis=(2, 4))
    _, idx = lax.top_k(score.reshape(B, nWin), nWF)
    idx = (idx + jnp.arange(B, dtype=idx.dtype)[:, None] * nWin).reshape(-1)
    S = B * nWF
    Sp = ((S + T - 1) // T) * T
    idx_p = jnp.concatenate(
        [idx, jnp.broadcast_to(idx[-1:], (Sp - S,))]) if Sp > S else idx
    idx_p = idx_p.astype(jnp.int32)

    x_win = jnp.swapaxes(
        x_nhwc.reshape(B, nH, wsz, nW, wsz, Cc), 2, 3).reshape(
            B * nWin, winsz, Cc)

    def win_map(i, idxr, j=0):
        return (idxr[i * T + j], 0, 0)

    win_specs = [
        pl.BlockSpec((1, winsz, Cc), functools.partial(win_map, j=j))
        for j in range(T)
    ]
    C2 = Cc * CR
    hd = C2 // NUM_HEADS

    def pad_cols(w):
        parts = []
        for hh in range(NUM_HEADS):
            parts.append(w[..., hh * hd:(hh + 1) * hd])
            parts.append(jnp.zeros(w.shape[:-1] + (PH - hd,), w.dtype))
        return jnp.concatenate(parts, axis=-1)

    def pad_rows(w):
        parts = []
        for hh in range(NUM_HEADS):
            parts.append(w[hh * hd:(hh + 1) * hd, :])
            parts.append(jnp.zeros((PH - hd,) + w.shape[1:], w.dtype))
        return jnp.concatenate(parts, axis=0)

    lin_wp = _bf(pad_cols(p['lin_w']))
    lin_bp = pad_cols(p['lin_b']).reshape(1, PC2)
    qw = p['qkv_w']
    qkv_wp = _bf(jnp.concatenate(
        [pad_rows(pad_cols(qw[:, t * C2:(t + 1) * C2])) for t in range(3)],
        axis=1))
    qb = p['qkv_b']
    qkv_bp = jnp.concatenate(
        [pad_cols(qb[t * C2:(t + 1) * C2]) for t in range(3)]).reshape(
            1, 3 * PC2)
    proj_wp = _bf(pad_rows(p['proj_w']))

    def const2(i, idxr):
        return (0, 0)

    wt_specs = [
        pl.BlockSpec((1, Cc), const2),
        pl.BlockSpec((1, Cc), const2),
        pl.BlockSpec((Cc, PC2), const2),
        pl.BlockSpec((1, PC2), const2),
        pl.BlockSpec((PC2, 3 * PC2), const2),
        pl.BlockSpec((1, 3 * PC2), const2),
        pl.BlockSpec((PC2, Cc), const2),
        pl.BlockSpec((1, Cc), const2),
    ]
    compact = pl.pallas_call(
        _compute_body,
        grid_spec=pltpu.PrefetchScalarGridSpec(
            num_scalar_prefetch=1,
            grid=(Sp // T,),
            in_specs=win_specs + wt_specs,
            out_specs=pl.BlockSpec((T * winsz, Cc), lambda i, idxr: (i, 0)),
        ),
        out_shape=jax.ShapeDtypeStruct((Sp * winsz, Cc), jnp.float32),
    )(idx_p, *([x_win] * T),
      p['norm_g'].reshape(1, Cc), p['norm_b'].reshape(1, Cc),
      lin_wp, lin_bp, qkv_wp, qkv_bp,
      proj_wp, p['proj_b'].reshape(1, Cc))

    x_win_new = pl.pallas_call(
        _scatter_body,
        grid_spec=pltpu.PrefetchScalarGridSpec(
            num_scalar_prefetch=1,
            grid=(S,),
            in_specs=[
                pl.BlockSpec((winsz, Cc), lambda i, idxr: (i, 0)),
                pl.BlockSpec(memory_space=pl.ANY),
            ],
            out_specs=pl.BlockSpec(
                (1, winsz, Cc), lambda i, idxr: (idxr[i], 0, 0)),
        ),
        out_shape=jax.ShapeDtypeStruct(x_win.shape, jnp.float32),
        input_output_aliases={2: 0},
    )(idx_p, compact, x_win)

    x_nhwc = jnp.swapaxes(
        x_win_new.reshape(B, nH, nW, wsz, wsz, Cc), 2, 3).reshape(
            B, H, W, Cc)
    if shift_direct == 1:
        x_nhwc = x_nhwc[:, :, s2:-s2, :]
    elif shift_direct == 2:
        x_nhwc = x_nhwc[:, s2:-s2, :, :]
    elif shift_direct == 3:
        x_nhwc = x_nhwc[:, s2:-s2, s2:-s2, :]
    return x_nhwc


def _sep_conv_bn_relu6_nhwc(x, dw, pw, g, b, m, v):
    dwt = jnp.transpose(dw, (2, 3, 1, 0))
    pwt = jnp.transpose(pw, (2, 3, 1, 0))
    y = lax.conv_general_dilated(
        x, dwt, (1, 1), [(3, 3), (3, 3)],
        dimension_numbers=('NHWC', 'HWIO', 'NHWC'),
        feature_group_count=x.shape[-1])
    y = lax.conv_general_dilated(
        y, pwt, (1, 1), [(0, 0), (0, 0)],
        dimension_numbers=('NHWC', 'HWIO', 'NHWC'))
    y = (y - m) * lax.rsqrt(v + 1e-5)
    y = y * g + b
    return jnp.clip(y, 0.0, 6.0)


def kernel(feature_map, coarse_pred, params):
    x = jnp.transpose(feature_map, (0, 2, 3, 1))
    x = _sep_conv_bn_relu6_nhwc(
        x, params['conv_in_dw'], params['conv_in_pw'],
        params['conv_in_bn_g'], params['conv_in_bn_b'],
        params['conv_in_bn_m'], params['conv_in_bn_v'])
    probs = jax.nn.softmax(coarse_pred, axis=1)
    unc = 1.0 - probs.max(axis=1)
    for i in range(NUM_BLOCKS):
        if i != 0:
            x = _sep_conv_bn_relu6_nhwc(
                x, params['conv_%d_dw' % i], params['conv_%d_pw' % i],
                params['conv_%d_bn_g' % i], params['conv_%d_bn_b' % i],
                params['conv_%d_bn_m' % i], params['conv_%d_bn_v' % i])
        pre = 'blk%d_' % i
        p = {k[len(pre):]: v for k, v in params.items() if k.startswith(pre)}
        x = _refine_block_pallas(x, unc, i % 4, p)
    return jnp.transpose(x, (0, 3, 1, 2))
